# baseline (speedup 1.0000x reference)
import functools

import jax
import jax.numpy as jnp
from jax import lax
from jax.experimental import pallas as pl
from jax.experimental.pallas import tpu as pltpu
from jax.experimental.pallas import tpu_sc as plsc

EMBED_DIM = 32
PAD_DIM = 128
WINDOW = 256
LANES = 16
UNROLL = 8
NUM_CORES = 2
NUM_SUBCORES = 16
NUM_WORKERS = NUM_CORES * NUM_SUBCORES


def kernel(token_ids, table):
    batch, seq = token_ids.shape
    n = batch * seq
    per_worker = n // NUM_WORKERS
    assert per_worker * NUM_WORKERS == n and per_worker % WINDOW == 0
    steps = per_worker // WINDOW
    assert steps % 2 == 0
    idx = token_ids.reshape(n)
    table128 = jnp.pad(table, ((0, 0), (0, PAD_DIM - EMBED_DIM)))

    mesh = plsc.VectorSubcoreMesh(core_axis_name="c", subcore_axis_name="s")

    @functools.partial(
        pl.kernel,
        out_type=jax.ShapeDtypeStruct((batch, seq, EMBED_DIM), table.dtype),
        mesh=mesh,
        scratch_types=[
            pltpu.VMEM((WINDOW,), jnp.int32),
            pltpu.VMEM((WINDOW,), jnp.int32),
            pltpu.VMEM((WINDOW, PAD_DIM), jnp.float32),
            pltpu.VMEM((WINDOW, PAD_DIM), jnp.float32),
            pltpu.VMEM((WINDOW, EMBED_DIM), jnp.float32),
            pltpu.SemaphoreType.DMA,
            pltpu.SemaphoreType.DMA,
        ],
    )
    def run(table_hbm, idx_hbm, out_hbm3, i0, i1, r0, r1, packed_v, s0, s1):
        out_hbm = out_hbm3.reshape(n, EMBED_DIM)
        wid = lax.axis_index("s") * NUM_CORES + lax.axis_index("c")
        base = wid * per_worker
        idx_bufs = (i0, i1)
        row_bufs = (r0, r1)
        sems = (s0, s1)

        def fetch(w, b):
            iv, rv, sem = idx_bufs[b], row_bufs[b], sems[b]
            pltpu.sync_copy(idx_hbm.at[pl.ds(base + w * WINDOW, WINDOW)], iv)

            @pl.loop(0, WINDOW, step=LANES)
            def _(c):
                slc = pl.ds(c, LANES)
                iv.at[slc][...] = jnp.maximum(iv.at[slc][...], 0)

            pltpu.async_copy(table_hbm.at[iv], rv, sem)

        def finish(w, b):
            iv, rv, sem = idx_bufs[b], row_bufs[b], sems[b]
            pltpu.make_async_copy(table_hbm.at[iv], rv, sem).wait()

            @pl.loop(0, WINDOW, step=UNROLL)
            def _(i):
                for u in range(UNROLL):
                    for h in range(EMBED_DIM // LANES):
                        slc = (pl.ds(i + u, 1), pl.ds(h * LANES, LANES))
                        packed_v.at[*slc][...] = rv.at[*slc][...]

            pltpu.sync_copy(
                packed_v, out_hbm.at[pl.ds(base + w * WINDOW, WINDOW)]
            )

        fetch(0, 0)

        @pl.loop(0, steps // 2)
        def _(p):
            w0 = 2 * p
            fetch(w0 + 1, 1)
            finish(w0, 0)

            @pl.when(w0 + 2 < steps)
            def _():
                fetch(w0 + 2, 0)

            finish(w0 + 1, 1)

    return run(table128, idx)


# speedup vs baseline: 1.1233x; 1.1233x over previous
import functools

import jax
import jax.numpy as jnp
from jax import lax
from jax.experimental import pallas as pl
from jax.experimental.pallas import tpu as pltpu
from jax.experimental.pallas import tpu_sc as plsc

EMBED_DIM = 32
PAD_DIM = 128
WINDOW = 320
LANES = 16
UNROLL = 8
NUM_CORES = 2
NUM_SUBCORES = 16
NUM_WORKERS = NUM_CORES * NUM_SUBCORES


def kernel(token_ids, table):
    batch, seq = token_ids.shape
    n = batch * seq
    per_worker = n // NUM_WORKERS
    assert per_worker * NUM_WORKERS == n and per_worker % WINDOW == 0
    steps = per_worker // WINDOW
    assert steps % 2 == 0
    idx = token_ids.reshape(n)
    table128 = jnp.pad(table, ((0, 0), (0, PAD_DIM - EMBED_DIM)))

    mesh = plsc.VectorSubcoreMesh(core_axis_name="c", subcore_axis_name="s")

    @functools.partial(
        pl.kernel,
        out_type=jax.ShapeDtypeStruct((n, EMBED_DIM), table.dtype),
        mesh=mesh,
        scratch_types=[
            pltpu.VMEM((WINDOW,), jnp.int32),
            pltpu.VMEM((WINDOW,), jnp.int32),
            pltpu.VMEM((WINDOW, PAD_DIM), jnp.float32),
            pltpu.VMEM((WINDOW, PAD_DIM), jnp.float32),
            pltpu.VMEM((WINDOW, EMBED_DIM), jnp.float32),
            pltpu.SemaphoreType.DMA,
            pltpu.SemaphoreType.DMA,
        ],
    )
    def run(table_hbm, idx_hbm, out_hbm, i0, i1, r0, r1, packed_v, s0, s1):
        wid = lax.axis_index("s") * NUM_CORES + lax.axis_index("c")
        base = wid * per_worker
        idx_bufs = (i0, i1)
        row_bufs = (r0, r1)
        sems = (s0, s1)

        def fetch(w, b):
            iv, rv, sem = idx_bufs[b], row_bufs[b], sems[b]
            pltpu.sync_copy(idx_hbm.at[pl.ds(base + w * WINDOW, WINDOW)], iv)

            @pl.loop(0, WINDOW, step=LANES)
            def _(c):
                slc = pl.ds(c, LANES)
                iv.at[slc][...] = jnp.maximum(iv.at[slc][...], 0)

            pltpu.async_copy(table_hbm.at[iv], rv, sem)

        def finish(w, b):
            iv, rv, sem = idx_bufs[b], row_bufs[b], sems[b]
            pltpu.make_async_copy(table_hbm.at[iv], rv, sem).wait()

            @pl.loop(0, WINDOW, step=UNROLL)
            def _(i):
                for u in range(UNROLL):
                    for h in range(EMBED_DIM // LANES):
                        slc = (pl.ds(i + u, 1), pl.ds(h * LANES, LANES))
                        packed_v.at[*slc][...] = rv.at[*slc][...]

            pltpu.sync_copy(
                packed_v, out_hbm.at[pl.ds(base + w * WINDOW, WINDOW)]
            )

        fetch(0, 0)

        @pl.loop(0, steps // 2)
        def _(p):
            w0 = 2 * p
            fetch(w0 + 1, 1)
            finish(w0, 0)

            @pl.when(w0 + 2 < steps)
            def _():
                fetch(w0 + 2, 0)

            finish(w0 + 1, 1)

    out = run(table128, idx)
    return out.reshape(batch, seq, EMBED_DIM)
